# baseline (speedup 1.0000x reference)
import functools

import jax
import jax.numpy as jnp
from jax import lax
from jax.experimental import pallas as pl
from jax.experimental.pallas import tpu as pltpu
from jax.experimental.pallas import tpu_sc as plsc

_TM = 512
_TN = 1024


def _vq_body(z_ref, z2_ref, e2_ref, emb_ref, idx_ref, loss_ref, util_ref,
             pres_ref, acc_ref):
    i = pl.program_id(0)
    n_tiles = pl.num_programs(0)
    num_codes = emb_ref.shape[0]
    n_chunks = num_codes // _TN

    @pl.when(i == 0)
    def _init():
        pres_ref[...] = jnp.zeros_like(pres_ref)
        acc_ref[0] = jnp.float32(0.0)

    z = z_ref[...]
    z2 = z2_ref[...]
    best = jnp.full((_TM, 1), jnp.inf, jnp.float32)
    bidx = jnp.zeros((_TM, 1), jnp.int32)
    for j in range(n_chunks):
        e = emb_ref[pl.ds(j * _TN, _TN), :]
        e2c = e2_ref[0:1, pl.ds(j * _TN, _TN)]
        m = lax.dot_general(z, e, (((1,), (1,)), ((), ())),
                            preferred_element_type=jnp.float32)
        dist = (z2 + e2c) - m
        lmin = jnp.min(dist, axis=1, keepdims=True)
        col = lax.broadcasted_iota(jnp.int32, (_TM, _TN), 1) + j * _TN
        lidx = jnp.min(jnp.where(dist == lmin, col, jnp.int32(2**30)),
                       axis=1, keepdims=True)
        take = lmin < best
        best = jnp.where(take, lmin, best)
        bidx = jnp.where(take, lidx, bidx)

    idx_ref[...] = bidx.reshape(1, 1, _TM)
    acc_ref[0] += jnp.sum(best)

    for j in range(n_chunks):
        col = lax.broadcasted_iota(jnp.int32, (_TM, _TN), 1) + j * _TN
        hit = jnp.max((bidx == col).astype(jnp.float32), axis=0, keepdims=True)
        pres_ref[j:j + 1, :] = jnp.maximum(pres_ref[j:j + 1, :], hit)

    @pl.when(i == n_tiles - 1)
    def _finish():
        n_tok = n_tiles * _TM
        d = z_ref.shape[1]
        loss = acc_ref[0] / jnp.float32(n_tok * d)
        loss_ref[...] = jnp.broadcast_to(loss, (1, 1))
        util = jnp.sum(pres_ref[...]) / jnp.float32(num_codes)
        util_ref[...] = jnp.broadcast_to(util, (1, 1))


def _vq_argmin(zb, z2, e2, embedding):
    n_tok, d = zb.shape
    num_codes = embedding.shape[0]
    n_tiles = n_tok // _TM
    return pl.pallas_call(
        _vq_body,
        grid=(n_tiles,),
        in_specs=[
            pl.BlockSpec((_TM, d), lambda i: (i, 0)),
            pl.BlockSpec((_TM, 1), lambda i: (i, 0)),
            pl.BlockSpec((1, num_codes), lambda i: (0, 0)),
            pl.BlockSpec((num_codes, d), lambda i: (0, 0)),
        ],
        out_specs=[
            pl.BlockSpec((1, 1, _TM), lambda i: (i, 0, 0)),
            pl.BlockSpec((1, 1), lambda i: (0, 0)),
            pl.BlockSpec((1, 1), lambda i: (0, 0)),
        ],
        out_shape=[
            jax.ShapeDtypeStruct((n_tiles, 1, _TM), jnp.int32),
            jax.ShapeDtypeStruct((1, 1), jnp.float32),
            jax.ShapeDtypeStruct((1, 1), jnp.float32),
        ],
        scratch_shapes=[
            pltpu.VMEM((num_codes // _TN, _TN), jnp.float32),
            pltpu.SMEM((1,), jnp.float32),
        ],
        compiler_params=pltpu.CompilerParams(
            dimension_semantics=("arbitrary",),
            vmem_limit_bytes=100 * 1024 * 1024,
        ),
    )(zb, z2, e2, embedding)


def _sc_gather(embedding, idx_flat):
    n_tok = idx_flat.shape[0]
    d = embedding.shape[1]
    info = plsc.get_sparse_core_info()
    nc, ns = info.num_cores, info.num_subcores
    nw = nc * ns
    b_per_w = n_tok // nw
    n_sub = b_per_w // 128
    idx2d = idx_flat.reshape(n_tok // 128, 128)
    mesh = plsc.VectorSubcoreMesh(core_axis_name="c", subcore_axis_name="s")

    @functools.partial(
        pl.kernel, mesh=mesh,
        out_type=jax.ShapeDtypeStruct((n_tok, d), jnp.float32),
        scratch_types=[
            pltpu.VMEM((n_sub, 128), jnp.int32),
            pltpu.VMEM((b_per_w, d), jnp.float32),
            pltpu.SemaphoreType.DMA,
        ],
    )
    def gather_k(table_hbm, idx_hbm, out_hbm, idx_v, rows_v, sem):
        wid = lax.axis_index("s") * nc + lax.axis_index("c")
        pltpu.sync_copy(idx_hbm.at[pl.ds(wid * n_sub, n_sub)], idx_v)
        copies = [
            pltpu.async_copy(table_hbm.at[idx_v.at[j]],
                             rows_v.at[pl.ds(j * 128, 128)], sem)
            for j in range(n_sub)
        ]
        for c in copies:
            c.wait()
        pltpu.sync_copy(rows_v, out_hbm.at[pl.ds(wid * b_per_w, b_per_w)])

    return gather_k(embedding, idx2d)


def kernel(z_e, embedding):
    b, d, t = z_e.shape
    z_flat = jnp.transpose(z_e, (0, 2, 1)).reshape(b * t, d)
    z2 = jnp.sum(z_flat ** 2, axis=1, keepdims=True)
    e2 = jnp.sum(embedding ** 2, axis=1)[None, :]
    zb = (z_flat * 2.0).astype(jnp.bfloat16)
    idx3, loss11, util11 = _vq_argmin(zb, z2, e2, embedding)
    idx_flat = idx3.reshape(b * t)
    zq_flat = _sc_gather(embedding, idx_flat)
    z_q = jnp.transpose(zq_flat.reshape(b, t, d), (0, 2, 1))
    indices = idx_flat.reshape(b, t)
    return z_q, indices, loss11[0, 0], util11[0, 0]
